# baseline (device time: 220305 ns/iter reference)
import jax
import jax.numpy as jnp
from jax import lax
from jax.experimental import pallas as pl
from jax.experimental.pallas import tpu as pltpu

N_DEV = 32
E_PER = 4
N_STEPS = N_DEV - 1


def kernel(x, router_W, route_idx, expert_W):
    n_tok, d_model = x.shape
    e_per, _, d_out = expert_W.shape
    chunk = n_tok // N_DEV

    def body(x_ref, rw_ref, idx_ref, w_ref, out_ref, comm_ref, send_sems, recv_sems):
        my = lax.axis_index("i")
        left = lax.rem(my + N_DEV - 1, N_DEV)
        right = lax.rem(my + 1, N_DEV)

        barrier_sem = pltpu.get_barrier_semaphore()
        for nbr in (left, right):
            pl.semaphore_signal(
                barrier_sem, inc=1,
                device_id=(nbr,), device_id_type=pl.DeviceIdType.MESH,
            )
        pl.semaphore_wait(barrier_sem, 2)

        idx = idx_ref[:, :]
        acc = jnp.zeros((n_tok, d_out), jnp.float32)
        for j in range(E_PER):
            e = my * E_PER + j
            mask = (idx == e).astype(jnp.float32)
            acc = acc + jnp.dot(
                x_ref[:, :] * mask, w_ref[j],
                preferred_element_type=jnp.float32,
            )
        out_ref[:, :] = acc

        for s in range(N_STEPS):
            send_idx = lax.rem(my - s + 2 * N_DEV, N_DEV)
            recv_idx = lax.rem(my - s - 1 + 2 * N_DEV, N_DEV)
            rdma = pltpu.make_async_remote_copy(
                src_ref=out_ref.at[pl.ds(send_idx * chunk, chunk), :],
                dst_ref=comm_ref.at[s],
                send_sem=send_sems.at[s],
                recv_sem=recv_sems.at[s],
                device_id=(right,),
                device_id_type=pl.DeviceIdType.MESH,
            )
            rdma.start()
            rdma.wait()
            out_ref[pl.ds(recv_idx * chunk, chunk), :] = (
                out_ref[pl.ds(recv_idx * chunk, chunk), :] + comm_ref[s]
            )

        for s in range(N_STEPS):
            send_idx = lax.rem(my + 1 - s + 2 * N_DEV, N_DEV)
            rdma = pltpu.make_async_remote_copy(
                src_ref=out_ref.at[pl.ds(send_idx * chunk, chunk), :],
                dst_ref=out_ref.at[pl.ds(send_idx * chunk, chunk), :],
                send_sem=send_sems.at[N_STEPS + s],
                recv_sem=recv_sems.at[N_STEPS + s],
                device_id=(right,),
                device_id_type=pl.DeviceIdType.MESH,
            )
            rdma.start()
            rdma.wait()

    return pl.pallas_call(
        body,
        out_shape=jax.ShapeDtypeStruct((n_tok, d_out), jnp.float32),
        in_specs=[
            pl.BlockSpec(memory_space=pltpu.VMEM),
            pl.BlockSpec(memory_space=pltpu.VMEM),
            pl.BlockSpec(memory_space=pltpu.VMEM),
            pl.BlockSpec(memory_space=pltpu.VMEM),
        ],
        out_specs=pl.BlockSpec(memory_space=pltpu.VMEM),
        scratch_shapes=[
            pltpu.VMEM((N_STEPS, chunk, d_out), jnp.float32),
            pltpu.SemaphoreType.DMA((2 * N_STEPS,)),
            pltpu.SemaphoreType.DMA((2 * N_STEPS,)),
        ],
        compiler_params=pltpu.CompilerParams(collective_id=0),
    )(x, router_W, route_idx, expert_W)


# device time: 126995 ns/iter; 1.7348x vs baseline; 1.7348x over previous
import jax
import jax.numpy as jnp
from jax import lax
from jax.experimental import pallas as pl
from jax.experimental.pallas import tpu as pltpu

N_DEV = 32
E_PER = 4


def kernel(x, router_W, route_idx, expert_W):
    n_tok, d_model = x.shape
    e_per, _, d_out = expert_W.shape
    chunk = n_tok // N_DEV

    def body(x_ref, rw_ref, idx_ref, w_ref, out_ref,
             comm_ref, send_sems, recv1_sems, recv2_sems):
        my = lax.axis_index("i")

        barrier_sem = pltpu.get_barrier_semaphore()
        for k in range(1, N_DEV):
            peer = lax.rem(my + k, N_DEV)
            pl.semaphore_signal(
                barrier_sem, inc=1,
                device_id=(peer,), device_id_type=pl.DeviceIdType.MESH,
            )
        pl.semaphore_wait(barrier_sem, N_DEV - 1)

        idx = idx_ref[:, :]
        xv = x_ref[:, :]
        acc = jnp.zeros((n_tok, d_out), jnp.float32)
        for j in range(E_PER):
            e = my * E_PER + j
            mask = (idx == e).astype(jnp.float32)
            acc = acc + jnp.dot(
                xv * mask, w_ref[j], preferred_element_type=jnp.float32,
            )
        out_ref[:, :] = acc

        sends = []
        for k in range(1, N_DEV):
            t = lax.rem(my + k, N_DEV)
            rdma = pltpu.make_async_remote_copy(
                src_ref=out_ref.at[pl.ds(t * chunk, chunk), :],
                dst_ref=comm_ref.at[my],
                send_sem=send_sems.at[t],
                recv_sem=recv1_sems.at[my],
                device_id=(t,),
                device_id_type=pl.DeviceIdType.MESH,
            )
            rdma.start()
            sends.append(rdma)

        comm_ref[pl.ds(my, 1)] = out_ref[pl.ds(my * chunk, chunk), :].reshape(
            1, chunk, d_out
        )

        for k in range(1, N_DEV):
            s = lax.rem(my + k, N_DEV)
            recv = pltpu.make_async_remote_copy(
                src_ref=comm_ref.at[s],
                dst_ref=comm_ref.at[s],
                send_sem=send_sems.at[s],
                recv_sem=recv1_sems.at[s],
                device_id=(s,),
                device_id_type=pl.DeviceIdType.MESH,
            )
            recv.wait_recv()

        out_ref[pl.ds(my * chunk, chunk), :] = jnp.sum(comm_ref[...], axis=0)

        for rdma in sends:
            rdma.wait_send()

        sends2 = []
        for k in range(1, N_DEV):
            t = lax.rem(my + k, N_DEV)
            rdma = pltpu.make_async_remote_copy(
                src_ref=out_ref.at[pl.ds(my * chunk, chunk), :],
                dst_ref=out_ref.at[pl.ds(my * chunk, chunk), :],
                send_sem=send_sems.at[t],
                recv_sem=recv2_sems.at[my],
                device_id=(t,),
                device_id_type=pl.DeviceIdType.MESH,
            )
            rdma.start()
            sends2.append(rdma)

        for k in range(1, N_DEV):
            s = lax.rem(my + k, N_DEV)
            recv = pltpu.make_async_remote_copy(
                src_ref=out_ref.at[pl.ds(s * chunk, chunk), :],
                dst_ref=out_ref.at[pl.ds(s * chunk, chunk), :],
                send_sem=send_sems.at[s],
                recv_sem=recv2_sems.at[s],
                device_id=(s,),
                device_id_type=pl.DeviceIdType.MESH,
            )
            recv.wait_recv()

        for rdma in sends2:
            rdma.wait_send()

    return pl.pallas_call(
        body,
        out_shape=jax.ShapeDtypeStruct((n_tok, d_out), jnp.float32),
        in_specs=[
            pl.BlockSpec(memory_space=pltpu.VMEM),
            pl.BlockSpec(memory_space=pltpu.VMEM),
            pl.BlockSpec(memory_space=pltpu.VMEM),
            pl.BlockSpec(memory_space=pltpu.VMEM),
        ],
        out_specs=pl.BlockSpec(memory_space=pltpu.VMEM),
        scratch_shapes=[
            pltpu.VMEM((N_DEV, chunk, d_out), jnp.float32),
            pltpu.SemaphoreType.DMA((N_DEV,)),
            pltpu.SemaphoreType.DMA((N_DEV,)),
            pltpu.SemaphoreType.DMA((N_DEV,)),
        ],
        compiler_params=pltpu.CompilerParams(collective_id=0),
    )(x, router_W, route_idx, expert_W)


# device time: 77186 ns/iter; 2.8542x vs baseline; 1.6453x over previous
import jax
import jax.numpy as jnp
from jax import lax
from jax.experimental import pallas as pl
from jax.experimental.pallas import tpu as pltpu

N_DEV = 32
E_PER = 4
N_BLK = 4


def kernel(x, router_W, route_idx, expert_W):
    n_tok, d_model = x.shape
    e_per, _, d_out = expert_W.shape
    chunk = n_tok // N_DEV
    blk = n_tok // N_BLK
    cpb = N_DEV // N_BLK

    def body(x_ref, rw_ref, idx_ref, w_ref, out_ref,
             stage_ref, comm_ref, gather_ref,
             send_sems, recv1_sems, recv2_sems):
        my = lax.axis_index("i")

        barrier_sem = pltpu.get_barrier_semaphore()
        for k in range(1, N_DEV):
            peer = lax.rem(my + k, N_DEV)
            pl.semaphore_signal(
                barrier_sem, inc=1,
                device_id=(peer,), device_id_type=pl.DeviceIdType.MESH,
            )
        pl.semaphore_wait(barrier_sem, N_DEV - 1)

        p1_sends = []
        for b in range(N_BLK):
            rows = pl.ds(b * blk, blk)
            xb = x_ref[rows, :]
            mb = idx_ref[rows, :]
            acc = jnp.zeros((blk, d_out), jnp.float32)
            for j in range(E_PER):
                e = my * E_PER + j
                mask = (mb == e).astype(jnp.float32)
                acc = acc + jnp.dot(
                    xb * mask, w_ref[j], preferred_element_type=jnp.float32,
                )
            stage_ref[rows, :] = acc.astype(jnp.bfloat16)
            for c in range(b * cpb, (b + 1) * cpb):
                rdma = pltpu.make_async_remote_copy(
                    src_ref=stage_ref.at[pl.ds(c * chunk, chunk), :],
                    dst_ref=comm_ref.at[my],
                    send_sem=send_sems.at[c],
                    recv_sem=recv1_sems.at[my],
                    device_id=(c,),
                    device_id_type=pl.DeviceIdType.MESH,
                )
                p1_sends.append((c, rdma))

                @pl.when(my != c)
                def _(rdma=rdma):
                    rdma.start()

        comm_ref[pl.ds(my, 1)] = stage_ref[pl.ds(my * chunk, chunk), :].reshape(
            1, chunk, d_out
        )

        for k in range(1, N_DEV):
            s = lax.rem(my + k, N_DEV)
            recv = pltpu.make_async_remote_copy(
                src_ref=comm_ref.at[s],
                dst_ref=comm_ref.at[s],
                send_sem=send_sems.at[s],
                recv_sem=recv1_sems.at[s],
                device_id=(s,),
                device_id_type=pl.DeviceIdType.MESH,
            )
            recv.wait_recv()

        reduced = jnp.sum(comm_ref[...].astype(jnp.float32), axis=0)
        gather_ref[pl.ds(my, 1)] = reduced.astype(jnp.bfloat16).reshape(
            1, chunk, d_out
        )

        for c, rdma in p1_sends:
            @pl.when(my != c)
            def _(rdma=rdma):
                rdma.wait_send()

        sends2 = []
        for k in range(1, N_DEV):
            t = lax.rem(my + k, N_DEV)
            rdma = pltpu.make_async_remote_copy(
                src_ref=gather_ref.at[my],
                dst_ref=gather_ref.at[my],
                send_sem=send_sems.at[t],
                recv_sem=recv2_sems.at[my],
                device_id=(t,),
                device_id_type=pl.DeviceIdType.MESH,
            )
            rdma.start()
            sends2.append(rdma)

        for k in range(1, N_DEV):
            s = lax.rem(my + k, N_DEV)
            recv = pltpu.make_async_remote_copy(
                src_ref=gather_ref.at[s],
                dst_ref=gather_ref.at[s],
                send_sem=send_sems.at[s],
                recv_sem=recv2_sems.at[s],
                device_id=(s,),
                device_id_type=pl.DeviceIdType.MESH,
            )
            recv.wait_recv()

        out_ref[:, :] = gather_ref[...].reshape(n_tok, d_out).astype(jnp.float32)

        for rdma in sends2:
            rdma.wait_send()

    return pl.pallas_call(
        body,
        out_shape=jax.ShapeDtypeStruct((n_tok, d_out), jnp.float32),
        in_specs=[
            pl.BlockSpec(memory_space=pltpu.VMEM),
            pl.BlockSpec(memory_space=pltpu.VMEM),
            pl.BlockSpec(memory_space=pltpu.VMEM),
            pl.BlockSpec(memory_space=pltpu.VMEM),
        ],
        out_specs=pl.BlockSpec(memory_space=pltpu.VMEM),
        scratch_shapes=[
            pltpu.VMEM((n_tok, d_out), jnp.bfloat16),
            pltpu.VMEM((N_DEV, chunk, d_out), jnp.bfloat16),
            pltpu.VMEM((N_DEV, chunk, d_out), jnp.bfloat16),
            pltpu.SemaphoreType.DMA((N_DEV,)),
            pltpu.SemaphoreType.DMA((N_DEV,)),
            pltpu.SemaphoreType.DMA((N_DEV,)),
        ],
        compiler_params=pltpu.CompilerParams(collective_id=0),
    )(x, router_W, route_idx, expert_W)
